# baseline (speedup 1.0000x reference)
import functools

import numpy as np
import jax
import jax.numpy as jnp
from jax import lax
from jax.experimental import pallas as pl
from jax.experimental.pallas import tpu as pltpu
from jax.experimental.pallas import tpu_sc as plsc

N_FEAT = 32
TCOLS = 128
NC, NS = 2, 16
NW = NC * NS
CHUNK = 1024
KGRP = CHUNK // 128
SEG = KGRP * 8 * 128


def _coord_onehot_t():
    table = np.zeros((7, 32), dtype=np.int64)
    table[0, 0] = 1
    table[0, 31] = 2
    table[1, :2] = [3, 4]
    table[1, 26:] = [5, 6, 7, 8, 9, 10]
    table[2, :2] = [11, 12]
    table[2, 26:] = [13, 14, 15, 16, 17, 18]
    table[3, :2] = [19, 20]
    table[3, 16:] = np.arange(21, 37)
    table[4, :2] = [37, 38]
    table[4, 16:] = np.arange(39, 55)
    table[5, :] = np.arange(55, 87)
    table[6, :] = np.arange(87, 119)
    row_of = np.zeros(119, dtype=np.int64)
    col_of = np.zeros(119, dtype=np.int64)
    rr, cc = np.nonzero(table)
    for r, c in zip(rr, cc):
        v = table[r, c]
        row_of[v] = r
        col_of[v] = c
    row_of[0] = 0
    col_of[0] = 0
    m = np.zeros((40, TCOLS), dtype=np.float32)
    for z in range(119):
        m[row_of[z], z] += 0.5
        m[8 + col_of[z], z] += 0.5
    return m


_M_ONEHOT_T = _coord_onehot_t()


def _table_body(rwt_ref, cwt_ref, mt_ref, t_ref):
    def rscale_t(v):
        norm = jnp.sqrt(jnp.sum(v * v, axis=0, keepdims=True))
        return v * jnp.minimum(1.0, 1.0 / jnp.maximum(norm, 1e-12))

    stacked_t = jnp.concatenate(
        [rscale_t(rwt_ref[...]), rscale_t(cwt_ref[...])], axis=1)
    t_ref[...] = jnp.dot(stacked_t, mt_ref[...],
                         preferred_element_type=jnp.float32)


def _build_table_t(row_weight, col_weight):
    rwt = jnp.concatenate(
        [row_weight, jnp.zeros((1, N_FEAT), jnp.float32)], axis=0).T
    cwt = col_weight.T
    return pl.pallas_call(
        _table_body,
        out_shape=jax.ShapeDtypeStruct((N_FEAT, TCOLS), jnp.float32),
    )(rwt, cwt, jnp.asarray(_M_ONEHOT_T))


def _gather_body(t_hbm, zt_hbm, out_hbm,
                 tt_v, idx0, idx1, rows0, rows1,
                 si0, si1, ss0, ss1):
    c = lax.axis_index("c")
    s = lax.axis_index("s")
    wid = s * NC + c
    n_j = zt_hbm.shape[0]
    g_per_j = zt_hbm.shape[1] // CHUNK
    n_chunks = (n_j * g_per_j) // NW
    n2 = n_chunks // 2
    u0 = wid * n_chunks

    idx = (idx0, idx1)
    rows = (rows0, rows1)
    si = (si0, si1)
    ss = (ss0, ss1)

    pltpu.sync_copy(t_hbm, tt_v)

    def issue_idx(u, b):
        j = u // g_per_j
        g = lax.rem(u, g_per_j)
        pltpu.async_copy(zt_hbm.at[j, pl.ds(g * CHUNK, CHUNK)], idx[b], si[b])

    issue_idx(u0, 0)

    def body(u2, _):
        for b in range(2):
            u = u0 + u2 * 2 + b
            j = u // g_per_j
            g = lax.rem(u, g_per_j)
            pltpu.make_async_copy(
                zt_hbm.at[0, pl.ds(0, CHUNK)], idx[b], si[b]).wait()

            @pl.when(u2 >= 1)
            def _():
                for c2 in range(4):
                    pltpu.make_async_copy(
                        rows[b].at[pl.ds(0, SEG)],
                        out_hbm.at[0, c2, pl.ds(0, SEG)], ss[b]).wait()

            @pl.when(u2 * 2 + b + 1 < n_chunks)
            def _():
                issue_idx(u + 1, 1 - b)

            @plsc.parallel_loop(0, CHUNK // 16, unroll=3)
            def _lk(m):
                iv = idx[b][pl.ds(m * 16, 16)]
                base = (m // 8) * 1024 + (m % 8) * 16
                for f in range(N_FEAT):
                    vals = plsc.load_gather(tt_v, [iv + (f * 128)])
                    c2, sub = divmod(f, 8)
                    rows[b][pl.ds(base + c2 * SEG + sub * 128, 16)] = vals

            for c2 in range(4):
                pltpu.async_copy(
                    rows[b].at[pl.ds(c2 * SEG, SEG)],
                    out_hbm.at[j, c2, pl.ds(g * SEG, SEG)], ss[b])
        return 0

    lax.fori_loop(0, n2, body, 0)
    for b in range(2):
        for c2 in range(4):
            pltpu.make_async_copy(
                rows[b].at[pl.ds(0, SEG)],
                out_hbm.at[0, c2, pl.ds(0, SEG)], ss[b]).wait()


def _gather(table_t, zt):
    n_j = zt.shape[0]
    n_i = zt.shape[1]
    mesh = plsc.VectorSubcoreMesh(
        core_axis_name="c", subcore_axis_name="s",
        num_cores=NC, num_subcores=NS)
    fn = pl.kernel(
        _gather_body,
        out_type=jax.ShapeDtypeStruct((n_j, 4, (n_i // 128) * 8 * 128),
                                      jnp.float32),
        mesh=mesh,
        compiler_params=pltpu.CompilerParams(
            use_tc_tiling_on_sc=False, needs_layout_passes=False),
        scratch_types=[
            pltpu.VMEM((N_FEAT * TCOLS,), jnp.float32),
            pltpu.VMEM((CHUNK,), jnp.int32),
            pltpu.VMEM((CHUNK,), jnp.int32),
            pltpu.VMEM((4 * SEG,), jnp.float32),
            pltpu.VMEM((4 * SEG,), jnp.float32),
            pltpu.SemaphoreType.DMA,
            pltpu.SemaphoreType.DMA,
            pltpu.SemaphoreType.DMA,
            pltpu.SemaphoreType.DMA,
        ],
    )
    return fn(table_t, zt)


def kernel(Z, row_weight, col_weight):
    b0, b1 = Z.shape
    table_t = _build_table_t(row_weight, col_weight).reshape(-1)
    zt = Z.T.astype(jnp.int32)
    out3d = _gather(table_t, zt)
    out5d = out3d.reshape(b1, 4, b0 // 128, 8, 128)
    return out5d.transpose(2, 4, 0, 1, 3).reshape(b0, b1, N_FEAT)


# speedup vs baseline: 2.0410x; 2.0410x over previous
import functools

import numpy as np
import jax
import jax.numpy as jnp
from jax import lax
from jax.experimental import pallas as pl
from jax.experimental.pallas import tpu as pltpu
from jax.experimental.pallas import tpu_sc as plsc

N_FEAT = 32
TCOLS = 128
NC, NS = 2, 16
NW = NC * NS
CHUNK = 1024
KGRP = CHUNK // 128
SEG = KGRP * 8 * 128


def _coord_onehot_t():
    table = np.zeros((7, 32), dtype=np.int64)
    table[0, 0] = 1
    table[0, 31] = 2
    table[1, :2] = [3, 4]
    table[1, 26:] = [5, 6, 7, 8, 9, 10]
    table[2, :2] = [11, 12]
    table[2, 26:] = [13, 14, 15, 16, 17, 18]
    table[3, :2] = [19, 20]
    table[3, 16:] = np.arange(21, 37)
    table[4, :2] = [37, 38]
    table[4, 16:] = np.arange(39, 55)
    table[5, :] = np.arange(55, 87)
    table[6, :] = np.arange(87, 119)
    row_of = np.zeros(119, dtype=np.int64)
    col_of = np.zeros(119, dtype=np.int64)
    rr, cc = np.nonzero(table)
    for r, c in zip(rr, cc):
        v = table[r, c]
        row_of[v] = r
        col_of[v] = c
    row_of[0] = 0
    col_of[0] = 0
    m = np.zeros((40, TCOLS), dtype=np.float32)
    for z in range(119):
        m[row_of[z], z] += 0.5
        m[8 + col_of[z], z] += 0.5
    return m


_M_ONEHOT_T = _coord_onehot_t()


def _table_body(rwt_ref, cwt_ref, mt_ref, t_ref):
    def rscale_t(v):
        norm = jnp.sqrt(jnp.sum(v * v, axis=0, keepdims=True))
        return v * jnp.minimum(1.0, 1.0 / jnp.maximum(norm, 1e-12))

    stacked_t = jnp.concatenate(
        [rscale_t(rwt_ref[...]), rscale_t(cwt_ref[...])], axis=1)
    t_ref[...] = jnp.dot(stacked_t, mt_ref[...],
                         preferred_element_type=jnp.float32)


def _build_table_t(row_weight, col_weight):
    rwt = jnp.concatenate(
        [row_weight, jnp.zeros((1, N_FEAT), jnp.float32)], axis=0).T
    cwt = col_weight.T
    return pl.pallas_call(
        _table_body,
        out_shape=jax.ShapeDtypeStruct((N_FEAT, TCOLS), jnp.float32),
    )(rwt, cwt, jnp.asarray(_M_ONEHOT_T))


def _gather_body(t_hbm, zt_hbm, out_hbm,
                 tt_v, idx0, idx1, rows0, rows1,
                 si0, si1, ss0, ss1):
    c = lax.axis_index("c")
    s = lax.axis_index("s")
    wid = s * NC + c
    n_j = zt_hbm.shape[0]
    g_per_j = zt_hbm.shape[1] // CHUNK
    n_chunks = (n_j * g_per_j) // NW
    n2 = n_chunks // 2
    u0 = wid * n_chunks

    idx = (idx0, idx1)
    rows = (rows0, rows1)
    si = (si0, si1)
    ss = (ss0, ss1)

    pltpu.sync_copy(t_hbm, tt_v)

    def issue_idx(u, b):
        j = u // g_per_j
        g = lax.rem(u, g_per_j)
        pltpu.async_copy(zt_hbm.at[j, pl.ds(g * CHUNK, CHUNK)], idx[b], si[b])

    issue_idx(u0, 0)

    def body(u2, _):
        for b in range(2):
            u = u0 + u2 * 2 + b
            j = u // g_per_j
            g = lax.rem(u, g_per_j)
            pltpu.make_async_copy(
                zt_hbm.at[0, pl.ds(0, CHUNK)], idx[b], si[b]).wait()

            @pl.when(u2 >= 1)
            def _():
                for c2 in range(4):
                    pltpu.make_async_copy(
                        rows[b].at[pl.ds(0, SEG)],
                        out_hbm.at[0, c2, pl.ds(0, SEG)], ss[b]).wait()

            @pl.when(u2 * 2 + b + 1 < n_chunks)
            def _():
                issue_idx(u + 1, 1 - b)

            for c2 in range(4):
                @plsc.parallel_loop(0, CHUNK // 16, unroll=2)
                def _lk(m, c2=c2):
                    iv = idx[b][pl.ds(m * 16, 16)]
                    base = (m // 8) * 1024 + (m % 8) * 16
                    for sub in range(8):
                        f = c2 * 8 + sub
                        vals = plsc.load_gather(tt_v, [iv + (f * 128)])
                        rows[b][pl.ds(base + c2 * SEG + sub * 128, 16)] = vals

                pltpu.async_copy(
                    rows[b].at[pl.ds(c2 * SEG, SEG)],
                    out_hbm.at[j, c2, pl.ds(g * SEG, SEG)], ss[b])
        return 0

    lax.fori_loop(0, n2, body, 0)
    for b in range(2):
        for c2 in range(4):
            pltpu.make_async_copy(
                rows[b].at[pl.ds(0, SEG)],
                out_hbm.at[0, c2, pl.ds(0, SEG)], ss[b]).wait()


def _gather(table_t, zt):
    n_j = zt.shape[0]
    n_i = zt.shape[1]
    mesh = plsc.VectorSubcoreMesh(
        core_axis_name="c", subcore_axis_name="s",
        num_cores=NC, num_subcores=NS)
    fn = pl.kernel(
        _gather_body,
        out_type=jax.ShapeDtypeStruct((n_j, 4, (n_i // 128) * 8 * 128),
                                      jnp.float32),
        mesh=mesh,
        compiler_params=pltpu.CompilerParams(
            use_tc_tiling_on_sc=False, needs_layout_passes=False),
        scratch_types=[
            pltpu.VMEM((N_FEAT * TCOLS,), jnp.float32),
            pltpu.VMEM((CHUNK,), jnp.int32),
            pltpu.VMEM((CHUNK,), jnp.int32),
            pltpu.VMEM((4 * SEG,), jnp.float32),
            pltpu.VMEM((4 * SEG,), jnp.float32),
            pltpu.SemaphoreType.DMA,
            pltpu.SemaphoreType.DMA,
            pltpu.SemaphoreType.DMA,
            pltpu.SemaphoreType.DMA,
        ],
    )
    return fn(table_t, zt)


def kernel(Z, row_weight, col_weight):
    b0, b1 = Z.shape
    table_t = _build_table_t(row_weight, col_weight).reshape(-1)
    zt = Z.T.astype(jnp.int32)
    out3d = _gather(table_t, zt)
    out5d = out3d.reshape(b1, 4, b0 // 128, 8, 128)
    return out5d.transpose(2, 4, 0, 1, 3).reshape(b0, b1, N_FEAT)


# speedup vs baseline: 2.0613x; 1.0099x over previous
import functools

import numpy as np
import jax
import jax.numpy as jnp
from jax import lax
from jax.experimental import pallas as pl
from jax.experimental.pallas import tpu as pltpu
from jax.experimental.pallas import tpu_sc as plsc

N_FEAT = 32
TCOLS = 128
NC, NS = 2, 16
NW = NC * NS
CHUNK = 1024
KGRP = CHUNK // 128
SEG = KGRP * 8 * 128


def _coord_onehot_t():
    table = np.zeros((7, 32), dtype=np.int64)
    table[0, 0] = 1
    table[0, 31] = 2
    table[1, :2] = [3, 4]
    table[1, 26:] = [5, 6, 7, 8, 9, 10]
    table[2, :2] = [11, 12]
    table[2, 26:] = [13, 14, 15, 16, 17, 18]
    table[3, :2] = [19, 20]
    table[3, 16:] = np.arange(21, 37)
    table[4, :2] = [37, 38]
    table[4, 16:] = np.arange(39, 55)
    table[5, :] = np.arange(55, 87)
    table[6, :] = np.arange(87, 119)
    row_of = np.zeros(119, dtype=np.int64)
    col_of = np.zeros(119, dtype=np.int64)
    rr, cc = np.nonzero(table)
    for r, c in zip(rr, cc):
        v = table[r, c]
        row_of[v] = r
        col_of[v] = c
    row_of[0] = 0
    col_of[0] = 0
    m = np.zeros((40, TCOLS), dtype=np.float32)
    for z in range(119):
        m[row_of[z], z] += 0.5
        m[8 + col_of[z], z] += 0.5
    return m


_M_ONEHOT_T = _coord_onehot_t()


def _table_body(rwt_ref, cwt_ref, mt_ref, t_ref):
    def rscale_t(v):
        norm = jnp.sqrt(jnp.sum(v * v, axis=0, keepdims=True))
        return v * jnp.minimum(1.0, 1.0 / jnp.maximum(norm, 1e-12))

    stacked_t = jnp.concatenate(
        [rscale_t(rwt_ref[...]), rscale_t(cwt_ref[...])], axis=1)
    t_ref[...] = jnp.dot(stacked_t, mt_ref[...],
                         preferred_element_type=jnp.float32)


def _build_table_t(row_weight, col_weight):
    rwt = jnp.concatenate(
        [row_weight, jnp.zeros((1, N_FEAT), jnp.float32)], axis=0).T
    cwt = col_weight.T
    return pl.pallas_call(
        _table_body,
        out_shape=jax.ShapeDtypeStruct((N_FEAT, TCOLS), jnp.float32),
    )(rwt, cwt, jnp.asarray(_M_ONEHOT_T))


def _gather_body(t_hbm, zt_hbm, out_hbm,
                 tt_v, idx0, idx1, rows0, rows1,
                 si0, si1, ss0, ss1):
    c = lax.axis_index("c")
    s = lax.axis_index("s")
    wid = s * NC + c
    n_j = zt_hbm.shape[0]
    g_per_j = zt_hbm.shape[1] // CHUNK
    n_chunks = (n_j * g_per_j) // NW
    n2 = n_chunks // 2
    u0 = wid * n_chunks

    idx = (idx0, idx1)
    rows = (rows0, rows1)
    si = (si0, si1)
    ss = (ss0, ss1)

    pltpu.sync_copy(t_hbm, tt_v)

    def issue_idx(u, b):
        j = u // g_per_j
        g = lax.rem(u, g_per_j)
        pltpu.async_copy(zt_hbm.at[j, pl.ds(g * CHUNK, CHUNK)], idx[b], si[b])

    issue_idx(u0, 0)

    def body(u2, _):
        for b in range(2):
            u = u0 + u2 * 2 + b
            j = u // g_per_j
            g = lax.rem(u, g_per_j)
            pltpu.make_async_copy(
                zt_hbm.at[0, pl.ds(0, CHUNK)], idx[b], si[b]).wait()

            @pl.when(u2 >= 1)
            def _():
                for c2 in range(4):
                    pltpu.make_async_copy(
                        rows[b].at[pl.ds(0, SEG)],
                        out_hbm.at[0, c2, pl.ds(0, SEG)], ss[b]).wait()

            @pl.when(u2 * 2 + b + 1 < n_chunks)
            def _():
                issue_idx(u + 1, 1 - b)

            for c2 in range(4):
                @plsc.parallel_loop(0, CHUNK // 16, unroll=4)
                def _lk(m, c2=c2):
                    iv = idx[b][pl.ds(m * 16, 16)]
                    base = (m // 8) * 1024 + (m % 8) * 16
                    for sub in range(8):
                        f = c2 * 8 + sub
                        vals = plsc.load_gather(tt_v, [iv + (f * 128)])
                        rows[b][pl.ds(base + c2 * SEG + sub * 128, 16)] = vals

                pltpu.async_copy(
                    rows[b].at[pl.ds(c2 * SEG, SEG)],
                    out_hbm.at[j, c2, pl.ds(g * SEG, SEG)], ss[b])
        return 0

    lax.fori_loop(0, n2, body, 0)
    for b in range(2):
        for c2 in range(4):
            pltpu.make_async_copy(
                rows[b].at[pl.ds(0, SEG)],
                out_hbm.at[0, c2, pl.ds(0, SEG)], ss[b]).wait()


def _gather(table_t, zt):
    n_j = zt.shape[0]
    n_i = zt.shape[1]
    mesh = plsc.VectorSubcoreMesh(
        core_axis_name="c", subcore_axis_name="s",
        num_cores=NC, num_subcores=NS)
    fn = pl.kernel(
        _gather_body,
        out_type=jax.ShapeDtypeStruct((n_j, 4, (n_i // 128) * 8 * 128),
                                      jnp.float32),
        mesh=mesh,
        compiler_params=pltpu.CompilerParams(
            use_tc_tiling_on_sc=False, needs_layout_passes=False),
        scratch_types=[
            pltpu.VMEM((N_FEAT * TCOLS,), jnp.float32),
            pltpu.VMEM((CHUNK,), jnp.int32),
            pltpu.VMEM((CHUNK,), jnp.int32),
            pltpu.VMEM((4 * SEG,), jnp.float32),
            pltpu.VMEM((4 * SEG,), jnp.float32),
            pltpu.SemaphoreType.DMA,
            pltpu.SemaphoreType.DMA,
            pltpu.SemaphoreType.DMA,
            pltpu.SemaphoreType.DMA,
        ],
    )
    return fn(table_t, zt)


def kernel(Z, row_weight, col_weight):
    b0, b1 = Z.shape
    table_t = _build_table_t(row_weight, col_weight).reshape(-1)
    zt = Z.T.astype(jnp.int32)
    out3d = _gather(table_t, zt)
    out5d = out3d.reshape(b1, 4, b0 // 128, 8, 128)
    return out5d.transpose(2, 4, 0, 1, 3).reshape(b0, b1, N_FEAT)
